# Initial kernel scaffold; baseline (speedup 1.0000x reference)
import jax
import jax.numpy as jnp
from jax.experimental import pallas as pl


def kernel(pred_logits, target_labels, weights):
    raise NotImplementedError("write your pallas kernel here")


# speedup vs baseline: 27.1987x; 27.1987x over previous
import jax
import jax.numpy as jnp
from jax import lax
from jax.experimental import pallas as pl
from jax.experimental.pallas import tpu as pltpu

_B = 8
_N = 100000
_NPAD = 100096
_ROWS = _NPAD // 128
_K = int(0.8 * (_B * _N))


def _body(x_ref, t_ref, w_ref, sgt_ref, cnt_ref, tbits_ref, dice_ref, bits_scr):
    row = lax.broadcasted_iota(jnp.int32, (_ROWS, 128), 0)
    col = lax.broadcasted_iota(jnp.int32, (_ROWS, 128), 1)
    valid = (row * 128 + col) < _N
    for b in range(_B):
        x = x_ref[b]
        t = t_ref[b]
        w = w_ref[b]
        e = jnp.exp(-jnp.abs(x))
        bce = (jnp.maximum(x, 0.0) - x * t + jnp.log(1.0 + e)) * w
        bits_scr[b] = lax.bitcast_convert_type(bce, jnp.int32)
        probs = jnp.where(valid, 1.0 / (1.0 + jnp.exp(-x)), 0.0)
        dice_ref[0, b] = jnp.sum(probs * t)
        dice_ref[1, b] = jnp.sum(probs)
        dice_ref[2, b] = jnp.sum(t)

    kf = jnp.float32(_K)

    def search(i, tbits):
        cand = tbits | (jnp.int32(1) << (jnp.int32(30) - i))
        cnt = jnp.float32(0.0)
        for b in range(_B):
            cnt += jnp.sum(jnp.where(bits_scr[b] >= cand, 1.0, 0.0))
        return jnp.where(cnt >= kf, cand, tbits)

    tbits = lax.fori_loop(0, 31, search, jnp.int32(0))

    s_gt = jnp.float32(0.0)
    c_gt = jnp.float32(0.0)
    for b in range(_B):
        bits = bits_scr[b]
        vals = lax.bitcast_convert_type(bits, jnp.float32)
        gt = bits > tbits
        s_gt += jnp.sum(jnp.where(gt, vals, 0.0))
        c_gt += jnp.sum(jnp.where(gt, 1.0, 0.0))
    sgt_ref[0, 0] = s_gt
    cnt_ref[0, 0] = c_gt
    tbits_ref[0, 0] = tbits


def kernel(pred_logits, target_labels, weights):
    def prep(a):
        a = a.reshape(_B, _N)
        a = jnp.pad(a, ((0, 0), (0, _NPAD - _N)))
        return a.reshape(_B, _ROWS, 128)

    x, t, w = prep(pred_logits), prep(target_labels), prep(weights)

    sgt, cgt, tbits, dice = pl.pallas_call(
        _body,
        out_shape=(
            jax.ShapeDtypeStruct((1, 1), jnp.float32),
            jax.ShapeDtypeStruct((1, 1), jnp.float32),
            jax.ShapeDtypeStruct((1, 1), jnp.int32),
            jax.ShapeDtypeStruct((3, _B), jnp.float32),
        ),
        out_specs=(
            pl.BlockSpec(memory_space=pltpu.SMEM),
            pl.BlockSpec(memory_space=pltpu.SMEM),
            pl.BlockSpec(memory_space=pltpu.SMEM),
            pl.BlockSpec(memory_space=pltpu.SMEM),
        ),
        scratch_shapes=[pltpu.VMEM((_B, _ROWS, 128), jnp.int32)],
    )(x, t, w)

    t_val = lax.bitcast_convert_type(tbits[0, 0], jnp.float32)
    s_top = sgt[0, 0] + (jnp.float32(_K) - cgt[0, 0]) * t_val
    bce_loss = s_top / jnp.float32(_K)
    inter, sum_p, sum_t = dice[0], dice[1], dice[2]
    dice_score = (2.0 * inter + 1e-06) / (sum_p + sum_t + 1e-06)
    dice_loss = jnp.mean(jnp.log(jnp.cosh(1.0 - dice_score)))
    total = 1.0 * bce_loss + 10.0 * dice_loss
    return (total, lax.stop_gradient(bce_loss), lax.stop_gradient(dice_loss))
